# baseline (speedup 1.0000x reference)
import jax
import jax.numpy as jnp
from jax.experimental import pallas as pl

_T = 2048


def _chamfer_body(p1_ref, p2t_ref, out_ref):
    n, _ = p1_ref.shape
    _, m = p2t_ref.shape
    p2 = p2t_ref[...]
    y2 = jnp.sum(p2 * p2, axis=0, keepdims=True)
    p2m2 = p2 * -2.0

    def one_tile(i, row_acc, col_min):
        q = p1_ref[pl.ds(i * _T, _T), :]
        x2 = jnp.sum(q * q, axis=1, keepdims=True)
        xy2 = jax.lax.dot_general(q, p2m2, (((1,), (0,)), ((), ())),
                                  preferred_element_type=jnp.float32)
        s = (x2 + y2) + xy2
        row_min = jnp.min(s, axis=1, keepdims=True)
        row_acc = row_acc + jnp.sum(jnp.maximum(row_min, 0.0))
        col_min = jnp.minimum(col_min, jnp.min(s, axis=0, keepdims=True))
        return row_acc, col_min

    row_acc = jnp.float32(0.0)
    col_min = jnp.full((1, m), jnp.inf, dtype=jnp.float32)
    for i in range(n // _T):
        row_acc, col_min = one_tile(i, row_acc, col_min)
    out_ref[...] = jnp.broadcast_to(
        row_acc + jnp.sum(jnp.maximum(col_min, 0.0)), (1, 1))


def kernel(points1, points2):
    _, n, _ = points1.shape
    p1 = points1.reshape(n, 3)
    p2t = points2.reshape(points2.shape[1], 3).T
    out = pl.pallas_call(
        _chamfer_body,
        out_shape=jax.ShapeDtypeStruct((1, 1), jnp.float32),
    )(p1, p2t)
    return out[0, 0]


# speedup vs baseline: 1.0372x; 1.0372x over previous
import jax
import jax.numpy as jnp
from jax.experimental import pallas as pl

_T = 1024


def _chamfer_body(p1_ref, p2t_ref, out_ref):
    n, _ = p1_ref.shape
    _, m = p2t_ref.shape
    p2 = p2t_ref[...]
    y2 = jnp.sum(p2 * p2, axis=0, keepdims=True)
    p2m2 = p2 * -2.0

    def one_tile(i, row_acc, col_min):
        q = p1_ref[pl.ds(i * _T, _T), :]
        x2 = jnp.sum(q * q, axis=1, keepdims=True)
        xy2 = jax.lax.dot_general(q, p2m2, (((1,), (0,)), ((), ())),
                                  preferred_element_type=jnp.float32)
        s = (x2 + y2) + xy2
        row_min = jnp.min(s, axis=1, keepdims=True)
        row_acc = row_acc + jnp.sum(jnp.maximum(row_min, 0.0))
        col_min = jnp.minimum(col_min, jnp.min(s, axis=0, keepdims=True))
        return row_acc, col_min

    row_acc = jnp.float32(0.0)
    col_min = jnp.full((1, m), jnp.inf, dtype=jnp.float32)
    for i in range(n // _T):
        row_acc, col_min = one_tile(i, row_acc, col_min)
    out_ref[...] = jnp.broadcast_to(
        row_acc + jnp.sum(jnp.maximum(col_min, 0.0)), (1, 1))


def kernel(points1, points2):
    _, n, _ = points1.shape
    p1 = points1.reshape(n, 3)
    p2t = points2.reshape(points2.shape[1], 3).T
    out = pl.pallas_call(
        _chamfer_body,
        out_shape=jax.ShapeDtypeStruct((1, 1), jnp.float32),
    )(p1, p2t)
    return out[0, 0]
